# baseline (speedup 1.0000x reference)
import functools

import jax
import jax.numpy as jnp
from jax import lax
from jax.experimental import pallas as pl
from jax.experimental.pallas import tpu as pltpu
from jax.experimental.pallas import tpu_sc as plsc

N = 10000
E = 320000
F = 128
H = 128
C = 40
CP = 64

NC = 2
NS = 16
NW = NC * NS
EPW = E // NW
K = 80
CHUNKS = EPW // K
NP = 10240
RPT = NP // NS
DEGW = 16


def _make_deg():
    mesh = plsc.VectorSubcoreMesh(core_axis_name="c", subcore_axis_name="s",
                                  num_cores=NC, num_subcores=NS)

    @functools.partial(
        pl.kernel,
        out_type=jax.ShapeDtypeStruct((NC, NP, DEGW), jnp.float32),
        mesh=mesh,
        compiler_params=pltpu.CompilerParams(use_tc_tiling_on_sc=False),
        scratch_types=[
            pltpu.VMEM((CHUNKS, K), jnp.int32),
            pltpu.VMEM((K, DEGW), jnp.float32),
            pltpu.VMEM((K, DEGW), jnp.float32),
            pltpu.VMEM_SHARED((NP, DEGW), jnp.float32),
            pltpu.SemaphoreType.DMA,
        ],
    )
    def deg(d3_hbm, out_hbm, dv, ones, zeros, acc, sem):
        c = lax.axis_index("c")
        t = lax.axis_index("s")
        wid = c * NS + t

        pltpu.sync_copy(d3_hbm.at[wid], dv)

        def fill_row(r, carry):
            ones[r, pl.ds(0, 16)] = jnp.full((16,), 1.0, jnp.float32)
            zeros[r, pl.ds(0, 16)] = jnp.zeros((16,), jnp.float32)
            return carry

        lax.fori_loop(0, K, fill_row, 0)
        for m in range(RPT // K):
            pltpu.sync_copy(zeros, acc.at[pl.ds(t * RPT + m * K, K)])
        plsc.subcore_barrier()

        AHEAD = 16

        def fire(i):
            pltpu.async_copy(ones, acc.at[dv.at[i]], sem, add=True)

        def drain1():
            pltpu.make_async_copy(ones, acc.at[dv.at[0]], sem).wait()

        for i in range(AHEAD):
            fire(i)

        def body(i, carry):
            drain1()
            fire(i)
            return carry

        lax.fori_loop(AHEAD, CHUNKS, body, 0)
        for _ in range(AHEAD):
            drain1()

        plsc.subcore_barrier()
        pltpu.sync_copy(acc.at[pl.ds(t * RPT, RPT)],
                        out_hbm.at[c, pl.ds(t * RPT, RPT)])

    return deg


_deg = _make_deg()


def _make_scatter(D: int, KC: int = K, NBUF: int = 4, GA: int = 2,
                  SB: int = 25):
    G = D // 16
    NCH = EPW // KC
    NB = NCH // SB
    IPF_OFF = NBUF - GA
    mesh = plsc.VectorSubcoreMesh(core_axis_name="c", subcore_axis_name="s",
                                  num_cores=NC, num_subcores=NS)

    @functools.partial(
        pl.kernel,
        out_type=jax.ShapeDtypeStruct((NC, NP, D), jnp.float32),
        mesh=mesh,
        compiler_params=pltpu.CompilerParams(use_tc_tiling_on_sc=False),
        scratch_types=[
            [pltpu.VMEM((SB, KC), jnp.int32) for _ in range(2)],
            [pltpu.VMEM((SB, KC), jnp.int32) for _ in range(2)],
            [pltpu.VMEM((KC, D), jnp.float32) for _ in range(NBUF)],
            pltpu.VMEM_SHARED((NP, D), jnp.float32),
            [pltpu.SemaphoreType.DMA for _ in range(NBUF)],
            [pltpu.SemaphoreType.DMA for _ in range(NBUF)],
            [pltpu.SemaphoreType.DMA for _ in range(2)],
        ],
    )
    def scat(rows_hbm, s3_hbm, d3_hbm, out_hbm, svb, dvb, bufs, acc, gsem,
             ssem, isem):
        c = lax.axis_index("c")
        t = lax.axis_index("s")
        wid = c * NS + t

        def ipf_descs(q):
            b = q % 2
            blk = pl.ds(q * SB, SB)
            return (
                pltpu.make_async_copy(s3_hbm.at[wid, blk], svb[b], isem[b]),
                pltpu.make_async_copy(d3_hbm.at[wid, blk], dvb[b], isem[b]),
            )

        def ipf(q):
            for desc in ipf_descs(q):
                desc.start()

        def iwait(q):
            for desc in ipf_descs(q):
                desc.wait()

        def gdesc(i):
            q, r = divmod(i, SB)
            return pltpu.make_async_copy(rows_hbm.at[svb[q % 2].at[r]],
                                         bufs[i % NBUF], gsem[i % NBUF])

        def ascatter(i):
            q, r = divmod(i, SB)
            pltpu.async_copy(bufs[i % NBUF], acc.at[dvb[q % 2].at[r]],
                             ssem[i % NBUF], add=True)

        def sdrain(b):
            pltpu.make_async_copy(bufs[b], acc.at[dvb[0].at[0]],
                                  ssem[b]).wait()

        ipf(0)

        def zero_row(r, carry):
            for g in range(G):
                bufs[0][r, pl.ds(g * 16, 16)] = jnp.zeros((16,), jnp.float32)
            return carry

        lax.fori_loop(0, KC, zero_row, 0)
        for m in range(RPT // KC):
            pltpu.sync_copy(bufs[0], acc.at[pl.ds(t * RPT + m * KC, KC)])
        plsc.subcore_barrier()

        iwait(0)
        ipf(1)
        for j in range(GA):
            gdesc(j).start()
        for i in range(NCH):
            ig = i + GA
            if ig < NCH:
                if ig >= NBUF:
                    sdrain(ig % NBUF)
                if ig % SB == 0:
                    iwait(ig // SB)
                gdesc(ig).start()
            if i % SB == IPF_OFF and 2 <= i // SB + 1 < NB:
                ipf(i // SB + 1)
            gdesc(i).wait()
            ascatter(i)
        for i in range(NCH - NBUF, NCH):
            sdrain(i % NBUF)

        plsc.subcore_barrier()
        pltpu.sync_copy(acc.at[pl.ds(t * RPT, RPT)],
                        out_hbm.at[c, pl.ds(t * RPT, RPT)])

    return scat


K128 = 40
_scat128 = _make_scatter(H, KC=K128, NBUF=8, GA=4)
_scat64 = _make_scatter(CP, NBUF=8, GA=4)


def _tc_in(x_ref, W1_ref, hw_ref):
    x = x_ref[...]
    m = jnp.mean(x, axis=0)
    xc = x - m
    v = jnp.mean(xc * xc, axis=0)
    h = xc * lax.rsqrt(v + 1e-5)
    hw_ref[...] = jnp.dot(h, W1_ref[...], preferred_element_type=jnp.float32)


def _tc_scale(degp_ref, hw_ref, hws_ref, dis_ref):
    deg = degp_ref[0, :N, :1] + degp_ref[1, :N, :1] + 1.0
    dis = lax.rsqrt(jnp.maximum(deg, 1.0))
    dis_ref[...] = dis
    hws_ref[...] = dis * hw_ref[...]


def _tc_mid(p_ref, hw1_ref, dis_ref, b1_ref, W2_ref, hws2_ref, hw2_ref):
    P = p_ref[0, :N, :] + p_ref[1, :N, :]
    dis = dis_ref[...]
    h1 = jnp.maximum(dis * P + (dis * dis) * hw1_ref[...] + b1_ref[...], 0.0)
    hw2 = jnp.dot(h1, W2_ref[...], preferred_element_type=jnp.float32)
    hw2_ref[...] = hw2
    hws2_ref[...] = dis * hw2


def _tc_out(p_ref, hw2_ref, dis_ref, b2_ref, out_ref):
    P = p_ref[0, :N, :] + p_ref[1, :N, :]
    dis = dis_ref[...]
    full = dis * P + (dis * dis) * hw2_ref[...] + b2_ref[...]
    out_ref[...] = full[:, :C]


def kernel(x, nodeblocks, W1, b1, W2, b2):
    s3 = nodeblocks[0].reshape(NW, CHUNKS, K)
    d3 = nodeblocks[1].reshape(NW, CHUNKS, K)
    s3f = nodeblocks[0].reshape(NW, EPW // K128, K128)
    d3f = nodeblocks[1].reshape(NW, EPW // K128, K128)

    degp = _deg(d3)
    hw1 = pl.pallas_call(
        _tc_in,
        out_shape=jax.ShapeDtypeStruct((N, H), jnp.float32),
    )(x, W1)
    hws1, dis = pl.pallas_call(
        _tc_scale,
        out_shape=[
            jax.ShapeDtypeStruct((N, H), jnp.float32),
            jax.ShapeDtypeStruct((N, 1), jnp.float32),
        ],
    )(degp, hw1)

    p1 = _scat128(hws1, s3f, d3f)

    W2p = jnp.pad(W2, ((0, 0), (0, CP - C)))
    b2p = jnp.pad(b2, (0, CP - C))
    hws2, hw2 = pl.pallas_call(
        _tc_mid,
        out_shape=[
            jax.ShapeDtypeStruct((N, CP), jnp.float32),
            jax.ShapeDtypeStruct((N, CP), jnp.float32),
        ],
    )(p1, hw1, dis, b1, W2p)

    p2 = _scat64(hws2, s3, d3)

    out = pl.pallas_call(
        _tc_out,
        out_shape=jax.ShapeDtypeStruct((N, C), jnp.float32),
    )(p2, hw2, dis, b2p)
    return out


# speedup vs baseline: 1.0035x; 1.0035x over previous
import functools

import jax
import jax.numpy as jnp
from jax import lax
from jax.experimental import pallas as pl
from jax.experimental.pallas import tpu as pltpu
from jax.experimental.pallas import tpu_sc as plsc

N = 10000
E = 320000
F = 128
H = 128
C = 40
CP = 64

NC = 2
NS = 16
NW = NC * NS
EPW = E // NW
K = 80
CHUNKS = EPW // K
NP = 10240
RPT = NP // NS
DEGW = 16


def _make_deg():
    mesh = plsc.VectorSubcoreMesh(core_axis_name="c", subcore_axis_name="s",
                                  num_cores=NC, num_subcores=NS)

    @functools.partial(
        pl.kernel,
        out_type=jax.ShapeDtypeStruct((NC, NP, DEGW), jnp.float32),
        mesh=mesh,
        compiler_params=pltpu.CompilerParams(use_tc_tiling_on_sc=False),
        scratch_types=[
            pltpu.VMEM((CHUNKS, K), jnp.int32),
            pltpu.VMEM((K, DEGW), jnp.float32),
            pltpu.VMEM((K, DEGW), jnp.float32),
            pltpu.VMEM_SHARED((NP, DEGW), jnp.float32),
            pltpu.SemaphoreType.DMA,
        ],
    )
    def deg(d3_hbm, out_hbm, dv, ones, zeros, acc, sem):
        c = lax.axis_index("c")
        t = lax.axis_index("s")
        wid = c * NS + t

        pltpu.sync_copy(d3_hbm.at[wid], dv)

        def fill_row(r, carry):
            ones[r, pl.ds(0, 16)] = jnp.full((16,), 1.0, jnp.float32)
            zeros[r, pl.ds(0, 16)] = jnp.zeros((16,), jnp.float32)
            return carry

        lax.fori_loop(0, K, fill_row, 0)
        for m in range(RPT // K):
            pltpu.sync_copy(zeros, acc.at[pl.ds(t * RPT + m * K, K)])
        plsc.subcore_barrier()

        AHEAD = 16

        def fire(i):
            pltpu.async_copy(ones, acc.at[dv.at[i]], sem, add=True)

        def drain1():
            pltpu.make_async_copy(ones, acc.at[dv.at[0]], sem).wait()

        for i in range(AHEAD):
            fire(i)

        def body(i, carry):
            drain1()
            fire(i)
            return carry

        lax.fori_loop(AHEAD, CHUNKS, body, 0)
        for _ in range(AHEAD):
            drain1()

        plsc.subcore_barrier()
        pltpu.sync_copy(acc.at[pl.ds(t * RPT, RPT)],
                        out_hbm.at[c, pl.ds(t * RPT, RPT)])

    return deg


_deg = _make_deg()


def _make_scatter(D: int, KC: int = K, NBUF: int = 4, GA: int = 2,
                  SB: int = 25):
    G = D // 16
    NCH = EPW // KC
    NB = NCH // SB
    IPF_OFF = NBUF - GA
    mesh = plsc.VectorSubcoreMesh(core_axis_name="c", subcore_axis_name="s",
                                  num_cores=NC, num_subcores=NS)

    @functools.partial(
        pl.kernel,
        out_type=jax.ShapeDtypeStruct((NC, NP, D), jnp.float32),
        mesh=mesh,
        compiler_params=pltpu.CompilerParams(use_tc_tiling_on_sc=False),
        scratch_types=[
            [pltpu.VMEM((SB, KC), jnp.int32) for _ in range(2)],
            [pltpu.VMEM((SB, KC), jnp.int32) for _ in range(2)],
            [pltpu.VMEM((KC, D), jnp.float32) for _ in range(NBUF)],
            pltpu.VMEM_SHARED((NP, D), jnp.float32),
            [pltpu.SemaphoreType.DMA for _ in range(NBUF)],
            [pltpu.SemaphoreType.DMA for _ in range(NBUF)],
            [pltpu.SemaphoreType.DMA for _ in range(2)],
        ],
    )
    def scat(rows_hbm, s3_hbm, d3_hbm, out_hbm, svb, dvb, bufs, acc, gsem,
             ssem, isem):
        c = lax.axis_index("c")
        t = lax.axis_index("s")
        wid = c * NS + t

        def ipf_descs(q):
            b = q % 2
            blk = pl.ds(q * SB, SB)
            return (
                pltpu.make_async_copy(s3_hbm.at[wid, blk], svb[b], isem[b]),
                pltpu.make_async_copy(d3_hbm.at[wid, blk], dvb[b], isem[b]),
            )

        def ipf(q):
            for desc in ipf_descs(q):
                desc.start()

        def iwait(q):
            for desc in ipf_descs(q):
                desc.wait()

        def gdesc(i):
            q, r = divmod(i, SB)
            return pltpu.make_async_copy(rows_hbm.at[svb[q % 2].at[r]],
                                         bufs[i % NBUF], gsem[i % NBUF])

        def ascatter(i):
            q, r = divmod(i, SB)
            pltpu.async_copy(bufs[i % NBUF], acc.at[dvb[q % 2].at[r]],
                             ssem[i % NBUF], add=True)

        def sdrain(b):
            pltpu.make_async_copy(bufs[b], acc.at[dvb[0].at[0]],
                                  ssem[b]).wait()

        ipf(0)

        def zero_row(r, carry):
            for g in range(G):
                bufs[0][r, pl.ds(g * 16, 16)] = jnp.zeros((16,), jnp.float32)
            return carry

        lax.fori_loop(0, KC, zero_row, 0)
        for m in range(RPT // KC):
            pltpu.sync_copy(bufs[0], acc.at[pl.ds(t * RPT + m * KC, KC)])
        plsc.subcore_barrier()

        iwait(0)
        ipf(1)
        for j in range(GA):
            gdesc(j).start()
        for i in range(NCH):
            ig = i + GA
            if ig < NCH:
                if ig >= NBUF:
                    sdrain(ig % NBUF)
                if ig % SB == 0:
                    iwait(ig // SB)
                gdesc(ig).start()
            if i % SB == IPF_OFF and 2 <= i // SB + 1 < NB:
                ipf(i // SB + 1)
            gdesc(i).wait()
            ascatter(i)
        for i in range(NCH - NBUF, NCH):
            sdrain(i % NBUF)

        plsc.subcore_barrier()
        pltpu.sync_copy(acc.at[pl.ds(t * RPT, RPT)],
                        out_hbm.at[c, pl.ds(t * RPT, RPT)])

    return scat


_scat128 = _make_scatter(H)
_scat64 = _make_scatter(CP, NBUF=8, GA=4)


def _tc_in(x_ref, W1_ref, hw_ref):
    x = x_ref[...]
    m = jnp.mean(x, axis=0)
    xc = x - m
    v = jnp.mean(xc * xc, axis=0)
    h = xc * lax.rsqrt(v + 1e-5)
    hw_ref[...] = jnp.dot(h, W1_ref[...], preferred_element_type=jnp.float32)


def _tc_scale(degp_ref, hw_ref, hws_ref, dis_ref):
    deg = degp_ref[0, :N, :1] + degp_ref[1, :N, :1] + 1.0
    dis = lax.rsqrt(jnp.maximum(deg, 1.0))
    dis_ref[...] = dis
    hws_ref[...] = dis * hw_ref[...]


def _tc_mid(p_ref, hw1_ref, dis_ref, b1_ref, W2_ref, hws2_ref, hw2_ref):
    P = p_ref[0, :N, :] + p_ref[1, :N, :]
    dis = dis_ref[...]
    h1 = jnp.maximum(dis * P + (dis * dis) * hw1_ref[...] + b1_ref[...], 0.0)
    hw2 = jnp.dot(h1, W2_ref[...], preferred_element_type=jnp.float32)
    hw2_ref[...] = hw2
    hws2_ref[...] = dis * hw2


def _tc_out(p_ref, hw2_ref, dis_ref, b2_ref, out_ref):
    P = p_ref[0, :N, :] + p_ref[1, :N, :]
    dis = dis_ref[...]
    full = dis * P + (dis * dis) * hw2_ref[...] + b2_ref[...]
    out_ref[...] = full[:, :C]


def kernel(x, nodeblocks, W1, b1, W2, b2):
    s3 = nodeblocks[0].reshape(NW, CHUNKS, K)
    d3 = nodeblocks[1].reshape(NW, CHUNKS, K)

    degp = _deg(d3)
    hw1 = pl.pallas_call(
        _tc_in,
        out_shape=jax.ShapeDtypeStruct((N, H), jnp.float32),
    )(x, W1)
    hws1, dis = pl.pallas_call(
        _tc_scale,
        out_shape=[
            jax.ShapeDtypeStruct((N, H), jnp.float32),
            jax.ShapeDtypeStruct((N, 1), jnp.float32),
        ],
    )(degp, hw1)

    p1 = _scat128(hws1, s3, d3)

    W2p = jnp.pad(W2, ((0, 0), (0, CP - C)))
    b2p = jnp.pad(b2, (0, CP - C))
    hws2, hw2 = pl.pallas_call(
        _tc_mid,
        out_shape=[
            jax.ShapeDtypeStruct((N, CP), jnp.float32),
            jax.ShapeDtypeStruct((N, CP), jnp.float32),
        ],
    )(p1, hw1, dis, b1, W2p)

    p2 = _scat64(hws2, s3, d3)

    out = pl.pallas_call(
        _tc_out,
        out_shape=jax.ShapeDtypeStruct((N, C), jnp.float32),
    )(p2, hw2, dis, b2p)
    return out
